# baseline (speedup 1.0000x reference)
import functools

import jax
import jax.numpy as jnp
from jax.experimental import pallas as pl
from jax.experimental.pallas import tpu as pltpu
from jax.experimental.pallas import tpu_sc as plsc

_NUM_TOKENS = 4096
_TOP_K = 2

_mesh = plsc.ScalarSubcoreMesh(axis_name="c", num_cores=1)


@functools.partial(
    pl.kernel,
    out_type=(
        jax.ShapeDtypeStruct((_NUM_TOKENS, _TOP_K), jnp.int32),
        jax.ShapeDtypeStruct((_NUM_TOKENS, _TOP_K), jnp.float32),
    ),
    mesh=_mesh,
    scratch_types=(
        pltpu.VMEM_SHARED((_NUM_TOKENS, _TOP_K), jnp.int32),
        pltpu.VMEM_SHARED((_NUM_TOKENS, _TOP_K), jnp.float32),
        pltpu.SemaphoreType.DMA,
        pltpu.SemaphoreType.DMA,
    ),
)
def _route_copy(rt_hbm, rs_hbm, out_rt, out_rs, rt_s, rs_s, sem_rt, sem_rs):
    c1 = pltpu.make_async_copy(rt_hbm, rt_s, sem_rt)
    c2 = pltpu.make_async_copy(rs_hbm, rs_s, sem_rs)
    c1.start()
    c2.start()
    c1.wait()
    c2.wait()
    c3 = pltpu.make_async_copy(rt_s, out_rt, sem_rt)
    c4 = pltpu.make_async_copy(rs_s, out_rs, sem_rs)
    c3.start()
    c4.start()
    c3.wait()
    c4.wait()


def kernel(router_logits, routing_tensor, routing_scales):
    del router_logits
    return _route_copy(routing_tensor, routing_scales)


# speedup vs baseline: 1.1153x; 1.1153x over previous
import functools

import jax
import jax.numpy as jnp
from jax import lax
from jax.experimental import pallas as pl
from jax.experimental.pallas import tpu as pltpu
from jax.experimental.pallas import tpu_sc as plsc

_NUM_TOKENS = 4096
_TOP_K = 2

_NW = 16
_ROWS_PER_W = _NUM_TOKENS // _NW

_mesh = plsc.VectorSubcoreMesh(
    core_axis_name="c", subcore_axis_name="s", num_cores=1
)


@functools.partial(
    pl.kernel,
    out_type=(
        jax.ShapeDtypeStruct((_NUM_TOKENS, _TOP_K), jnp.int32),
        jax.ShapeDtypeStruct((_NUM_TOKENS, _TOP_K), jnp.float32),
    ),
    mesh=_mesh,
    scratch_types=(
        pltpu.VMEM((_ROWS_PER_W, _TOP_K), jnp.int32),
        pltpu.VMEM((_ROWS_PER_W, _TOP_K), jnp.float32),
        pltpu.SemaphoreType.DMA,
        pltpu.SemaphoreType.DMA,
    ),
)
def _route_copy(rt_hbm, rs_hbm, out_rt, out_rs, rt_v, rs_v, sem_rt, sem_rs):
    wid = lax.axis_index("s")
    sl = pl.ds(wid * _ROWS_PER_W, _ROWS_PER_W)
    c1 = pltpu.make_async_copy(rt_hbm.at[sl], rt_v, sem_rt)
    c2 = pltpu.make_async_copy(rs_hbm.at[sl], rs_v, sem_rs)
    c1.start()
    c2.start()
    c1.wait()
    c2.wait()
    c3 = pltpu.make_async_copy(rt_v, out_rt.at[sl], sem_rt)
    c4 = pltpu.make_async_copy(rs_v, out_rs.at[sl], sem_rs)
    c3.start()
    c4.start()
    c3.wait()
    c4.wait()


def kernel(router_logits, routing_tensor, routing_scales):
    del router_logits
    return _route_copy(routing_tensor, routing_scales)


# speedup vs baseline: 2.4605x; 2.2061x over previous
import jax
import jax.numpy as jnp
from jax.experimental import pallas as pl

_NUM_TOKENS = 4096
_TOP_K = 2


def _copy_body(rt_ref, rs_ref, out_rt_ref, out_rs_ref):
    out_rt_ref[...] = rt_ref[...]
    out_rs_ref[...] = rs_ref[...]


_copy = pl.pallas_call(
    _copy_body,
    out_shape=(
        jax.ShapeDtypeStruct((_NUM_TOKENS, _TOP_K), jnp.int32),
        jax.ShapeDtypeStruct((_NUM_TOKENS, _TOP_K), jnp.float32),
    ),
)


def kernel(router_logits, routing_tensor, routing_scales):
    del router_logits
    return _copy(routing_tensor, routing_scales)
